# baseline (speedup 1.0000x reference)
import dataclasses

import jax
import jax.numpy as jnp
from jax import lax
from jax.experimental import pallas as pl
from jax.experimental.pallas import tpu as pltpu
from jax.experimental.pallas import tpu_sc as plsc
from jax._src.pallas import core as _pl_core

_GLOBAL_AVG = 3.5
_BATCH = 16384


def _on(mesh, mem_ref):
    return dataclasses.replace(
        mem_ref,
        memory_space=_pl_core.CoreMemorySpace(mem_ref.memory_space, mesh))


@jax.jit
def kernel(user_id, item_id, user_bias_score, item_bias_score):
    info = plsc.get_sparse_core_info()
    nc, ns, lanes = info.num_cores, info.num_subcores, info.num_lanes
    nw = nc * ns
    b_per_w = _BATCH // nw
    half = _BATCH // nc

    vmesh = plsc.VectorSubcoreMesh(core_axis_name="c", subcore_axis_name="s")
    smesh = plsc.ScalarSubcoreMesh(axis_name="c", num_cores=nc)

    def scs_fn(uid_hbm, iid_hbm, utab_hbm, itab_hbm, out_hbm,
               sidx_u, sidx_i, scs_sem_u, scs_sem_i, ready,
               uidx_v, iidx_v, uval_v, ival_v, sem_u, sem_i):
        c = lax.axis_index("c")
        cp_u = pltpu.async_copy(uid_hbm.at[pl.ds(c * half, half)], sidx_u,
                                scs_sem_u)
        cp_i = pltpu.async_copy(iid_hbm.at[pl.ds(c * half, half)], sidx_i,
                                scs_sem_i)
        cp_u.wait()
        cp_i.wait()
        pl.semaphore_signal(ready, 1, device_id={"s": 0})

    def tec_fn(uid_hbm, iid_hbm, utab_hbm, itab_hbm, out_hbm,
               sidx_u, sidx_i, scs_sem_u, scs_sem_i, ready,
               uidx_v, iidx_v, uval_v, ival_v, sem_u, sem_i):
        c = lax.axis_index("c")
        s = lax.axis_index("s")
        base = c * half + s * b_per_w
        loc = pl.ds(s * b_per_w, b_per_w)

        @pl.when(s == 0)
        def _():
            pl.semaphore_wait(ready, 1)
        plsc.subcore_barrier()
        pltpu.sync_copy(sidx_u.at[loc], uidx_v)
        g_u = pltpu.async_copy(utab_hbm.at[uidx_v], uval_v, sem_u)
        pltpu.sync_copy(sidx_i.at[loc], iidx_v)
        g_i = pltpu.async_copy(itab_hbm.at[iidx_v], ival_v, sem_i)
        g_u.wait()
        g_i.wait()

        @pl.loop(0, b_per_w, step=lanes)
        def _(j):
            sl = pl.ds(j, lanes)
            uval_v[sl] = uval_v[sl] + ival_v[sl] + _GLOBAL_AVG

        pltpu.sync_copy(uval_v, out_hbm.at[pl.ds(base, b_per_w)])

    run = pl.kernel(
        body=[tec_fn, scs_fn],
        mesh=[vmesh, smesh],
        out_type=jax.ShapeDtypeStruct((_BATCH,), jnp.float32),
        scratch_types=[
            pltpu.VMEM_SHARED((half,), jnp.int32),
            pltpu.VMEM_SHARED((half,), jnp.int32),
            _on(smesh, pltpu.SemaphoreType.DMA(())),
            _on(smesh, pltpu.SemaphoreType.DMA(())),
            _on(vmesh, pltpu.SemaphoreType.REGULAR(())),
            _on(vmesh, pltpu.VMEM((b_per_w,), jnp.int32)),
            _on(vmesh, pltpu.VMEM((b_per_w,), jnp.int32)),
            _on(vmesh, pltpu.VMEM((b_per_w,), jnp.float32)),
            _on(vmesh, pltpu.VMEM((b_per_w,), jnp.float32)),
            _on(vmesh, pltpu.SemaphoreType.DMA(())),
            _on(vmesh, pltpu.SemaphoreType.DMA(())),
        ],
    )
    return run(user_id, item_id, user_bias_score, item_bias_score)


# speedup vs baseline: 1.0052x; 1.0052x over previous
import dataclasses

import jax
import jax.numpy as jnp
from jax import lax
from jax.experimental import pallas as pl
from jax.experimental.pallas import tpu as pltpu
from jax.experimental.pallas import tpu_sc as plsc
from jax._src.pallas import core as _pl_core

_GLOBAL_AVG = 3.5
_BATCH = 16384


def _on(mesh, mem_ref):
    return dataclasses.replace(
        mem_ref,
        memory_space=_pl_core.CoreMemorySpace(mem_ref.memory_space, mesh))


@jax.jit
def kernel(user_id, item_id, user_bias_score, item_bias_score):
    info = plsc.get_sparse_core_info()
    nc, ns, lanes = info.num_cores, info.num_subcores, info.num_lanes
    nw = nc * ns
    b_per_w = _BATCH // nw
    half = _BATCH // nc

    vmesh = plsc.VectorSubcoreMesh(core_axis_name="c", subcore_axis_name="s")
    smesh = plsc.ScalarSubcoreMesh(axis_name="c", num_cores=nc)

    def scs_fn(uid_hbm, iid_hbm, utab_hbm, itab_hbm, out_hbm,
               sidx_u, sidx_i, scs_sem_u, scs_sem_i, ready,
               uidx_v, iidx_v, uval_v, ival_v, sem_u, sem_i):
        c = lax.axis_index("c")
        cp_u = pltpu.async_copy(uid_hbm.at[pl.ds(c * half, half)], sidx_u,
                                scs_sem_u)
        cp_i = pltpu.async_copy(iid_hbm.at[pl.ds(c * half, half)], sidx_i,
                                scs_sem_i)
        cp_u.wait()
        cp_i.wait()
        for t in range(ns):
            pl.semaphore_signal(ready, 1, device_id={"s": t})

    def tec_fn(uid_hbm, iid_hbm, utab_hbm, itab_hbm, out_hbm,
               sidx_u, sidx_i, scs_sem_u, scs_sem_i, ready,
               uidx_v, iidx_v, uval_v, ival_v, sem_u, sem_i):
        c = lax.axis_index("c")
        s = lax.axis_index("s")
        base = c * half + s * b_per_w
        loc = pl.ds(s * b_per_w, b_per_w)

        pl.semaphore_wait(ready, 1)
        pltpu.sync_copy(sidx_u.at[loc], uidx_v)
        g_u = pltpu.async_copy(utab_hbm.at[uidx_v], uval_v, sem_u)
        pltpu.sync_copy(sidx_i.at[loc], iidx_v)
        g_i = pltpu.async_copy(itab_hbm.at[iidx_v], ival_v, sem_i)
        g_u.wait()
        g_i.wait()

        @pl.loop(0, b_per_w, step=lanes)
        def _(j):
            sl = pl.ds(j, lanes)
            uval_v[sl] = uval_v[sl] + ival_v[sl] + _GLOBAL_AVG

        pltpu.sync_copy(uval_v, out_hbm.at[pl.ds(base, b_per_w)])

    run = pl.kernel(
        body=[tec_fn, scs_fn],
        mesh=[vmesh, smesh],
        out_type=jax.ShapeDtypeStruct((_BATCH,), jnp.float32),
        scratch_types=[
            pltpu.VMEM_SHARED((half,), jnp.int32),
            pltpu.VMEM_SHARED((half,), jnp.int32),
            _on(smesh, pltpu.SemaphoreType.DMA(())),
            _on(smesh, pltpu.SemaphoreType.DMA(())),
            _on(vmesh, pltpu.SemaphoreType.REGULAR(())),
            _on(vmesh, pltpu.VMEM((b_per_w,), jnp.int32)),
            _on(vmesh, pltpu.VMEM((b_per_w,), jnp.int32)),
            _on(vmesh, pltpu.VMEM((b_per_w,), jnp.float32)),
            _on(vmesh, pltpu.VMEM((b_per_w,), jnp.float32)),
            _on(vmesh, pltpu.SemaphoreType.DMA(())),
            _on(vmesh, pltpu.SemaphoreType.DMA(())),
        ],
    )
    return run(user_id, item_id, user_bias_score, item_bias_score)
